# baseline (speedup 1.0000x reference)
import functools

import jax
import jax.numpy as jnp
from jax import lax
from jax.experimental import pallas as pl
from jax.experimental.pallas import tpu as pltpu
from jax.experimental.pallas import tpu_sc as plsc

_NC, _NS = 2, 16
_CHUNK = 80

_NPG = 100
_INV = (1.0 + 1e-5) ** -0.5


def _fold(layer):
    s = layer["gamma"] * _INV
    return layer["W"] * s[None, :], layer["b"] * s + layer["beta"]


def _dot(a, w):
    return jax.lax.dot_general(
        a, w, (((1,), (0,)), ((), ())),
        preferred_element_type=jnp.float32,
        precision=jax.lax.Precision.DEFAULT,
    )


_STRIPE = 624


def _stripe_copy(s, src_ref, dst_ref, n):
    pltpu.sync_copy(src_ref.at[pl.ds(s * _STRIPE, _STRIPE)],
                    dst_ref.at[pl.ds(s * _STRIPE, _STRIPE)])
    tail = n - _STRIPE * _NS

    @pl.when(s == 0)
    def _():
        pltpu.sync_copy(src_ref.at[pl.ds(_STRIPE * _NS, tail)],
                        dst_ref.at[pl.ds(_STRIPE * _NS, tail)])


def _seg_sum_sc(table, src, dst, n, e):
    d = table.shape[1]
    nw = _NC * _NS
    epw = e // nw
    nch = epw // _CHUNK
    mesh = plsc.VectorSubcoreMesh(core_axis_name="c", subcore_axis_name="s")

    @functools.partial(
        pl.kernel, mesh=mesh,
        out_type=jax.ShapeDtypeStruct((_NC, n, d), jnp.float32),
        scratch_types=[
            pltpu.VMEM((_CHUNK,), jnp.int32),
            pltpu.VMEM((_CHUNK,), jnp.int32),
            pltpu.VMEM((_CHUNK, d), jnp.float32),
            pltpu.VMEM_SHARED((n, d), jnp.float32),
            pltpu.SemaphoreType.DMA,
        ])
    def k(table_hbm, src_hbm, dst_hbm, zeros_hbm, out_hbm,
          src_v, dst_v, rows_v, acc_sh, sem):
        c = lax.axis_index("c")
        s = lax.axis_index("s")
        w = c * _NS + s
        _stripe_copy(s, zeros_hbm, acc_sh, n)
        plsc.subcore_barrier()

        def body(i, carry):
            base = w * epw + i * _CHUNK
            pltpu.sync_copy(src_hbm.at[pl.ds(base, _CHUNK)], src_v)
            pltpu.sync_copy(dst_hbm.at[pl.ds(base, _CHUNK)], dst_v)
            pltpu.async_copy(table_hbm.at[src_v], rows_v, sem).wait()
            pltpu.sync_copy(rows_v, acc_sh.at[dst_v], add=True)
            return carry

        lax.fori_loop(0, nch, body, 0)
        plsc.subcore_barrier()
        _stripe_copy(s, acc_sh, out_hbm.at[c], n)

    return k(table, src, dst, jnp.zeros((n, d), jnp.float32))


def _edge_gather_sc(ab_tab, src, dst, e):
    d = ab_tab.shape[1]
    nw = _NC * _NS
    epw = e // nw
    nch = epw // _CHUNK
    mesh = plsc.VectorSubcoreMesh(core_axis_name="c", subcore_axis_name="s")

    @functools.partial(
        pl.kernel, mesh=mesh,
        out_type=(jax.ShapeDtypeStruct((e, d), jnp.float32),
                  jax.ShapeDtypeStruct((e, d), jnp.float32)),
        scratch_types=[
            pltpu.VMEM((_CHUNK,), jnp.int32),
            pltpu.VMEM((_CHUNK,), jnp.int32),
            pltpu.VMEM((_CHUNK, d), jnp.float32),
            pltpu.VMEM((_CHUNK, d), jnp.float32),
            pltpu.SemaphoreType.DMA,
        ])
    def k(ab_hbm, src_hbm, dst_hbm, oa_hbm, ob_hbm,
          di_v, si_v, ra_v, rb_v, sem):
        c = lax.axis_index("c")
        s = lax.axis_index("s")
        w = c * _NS + s

        def body(i, carry):
            base = w * epw + i * _CHUNK
            pltpu.sync_copy(dst_hbm.at[pl.ds(base, _CHUNK)], di_v)
            pltpu.sync_copy(src_hbm.at[pl.ds(base, _CHUNK)], si_v)
            cpa = pltpu.async_copy(ab_hbm.at[di_v], ra_v, sem)
            cpb = pltpu.async_copy(ab_hbm.at[si_v], rb_v, sem)
            cpa.wait()
            cpb.wait()
            pltpu.sync_copy(ra_v, oa_hbm.at[pl.ds(base, _CHUNK)])
            pltpu.sync_copy(rb_v, ob_hbm.at[pl.ds(base, _CHUNK)])
            return carry

        lax.fori_loop(0, nch, body, 0)

    return k(ab_tab, src, dst)


def _prep_body(x_ref, wab_ref, out_ref):
    out_ref[...] = _dot(x_ref[...], wab_ref[...])


def _edge_body(la_ref, lb_ref, b1_ref, w2_ref, b2_ref, w3_ref, b3_ref,
               out_ref):
    h = jnp.maximum(la_ref[:, :64] + lb_ref[:, 64:] + b1_ref[...], 0.0)
    h = jnp.maximum(_dot(h, w2_ref[...]) + b2_ref[...], 0.0)
    out_ref[...] = jnp.maximum(_dot(h, w3_ref[...]) + b3_ref[...], 0.0)


def _gin_body(h_ref, a0_ref, a1_ref, w_ref, b_ref, g_ref, t_ref, out_ref):
    din = h_ref.shape[1]
    u = h_ref[...] + a0_ref[:, :din] + a1_ref[:, :din]
    y = jnp.maximum(_dot(u, w_ref[...]) + b_ref[...], 0.0)
    out_ref[...] = g_ref[...] * y + t_ref[...]


def _lin_body(x0_ref, x1_ref, x2_ref, x3_ref, w0_ref, w1_ref, w2_ref,
              w3_ref, b_ref, out_ref):
    acc = _dot(x0_ref[...], w0_ref[...])
    acc += _dot(x1_ref[...], w1_ref[...])
    acc += _dot(x2_ref[...], w2_ref[...])
    acc += _dot(x3_ref[...], w3_ref[...])
    out_ref[...] = jnp.maximum(acc + b_ref[...], 0.0)


def _head_body(p_ref, w1_ref, b1_ref, w2_ref, b2_ref, wo_ref, bo_ref,
               out_ref):
    h = jnp.maximum(_dot(p_ref[...], w1_ref[...]) + b1_ref[...], 0.0)
    h = jnp.maximum(_dot(h, w2_ref[...]) + b2_ref[...], 0.0)
    z = _dot(h, wo_ref[...]) + bo_ref[...]
    zmax = jnp.max(z, axis=-1, keepdims=True)
    ez = jnp.exp(z - zmax)
    out_ref[...] = (z - zmax) - jnp.log(jnp.sum(ez, axis=-1, keepdims=True))


def _rows_call(body, n_rows, block_rows, feat_ins, bcast_ins, out_dim):
    grid = (n_rows // block_rows,)
    in_specs = (
        [pl.BlockSpec((block_rows, a.shape[-1]), lambda i: (i, 0))
         for a in feat_ins]
        + [pl.BlockSpec(a.shape, lambda i: tuple(0 for _ in a.shape))
           for a in bcast_ins]
    )
    return pl.pallas_call(
        body,
        grid=grid,
        in_specs=in_specs,
        out_specs=pl.BlockSpec((block_rows, out_dim), lambda i: (i, 0)),
        out_shape=jax.ShapeDtypeStruct((n_rows, out_dim), jnp.float32),
    )(*feat_ins, *bcast_ins)


def kernel(x, params, edge_index, batch):
    n = x.shape[0]
    e = edge_index.shape[1]
    b = n // _NPG
    src = edge_index[0]
    dst = edge_index[1]

    ec = [_fold(l) for l in params["edgeconv"]]
    w1, b1 = ec[0]
    wa = w1[:4] - w1[4:]
    wb = w1[4:]
    gin_w = [_fold(params[k][0]) for k in ("gin1", "gin2", "gin3")]
    bn_aff = [(params[k]["gamma"] * _INV, params[k]["beta"])
              for k in ("bn1", "bn2", "bn3")]
    wl, bl = _fold(params["lin"][0])
    wh1, bh1 = _fold(params["head1"][0])
    wh2, bh2 = _fold(params["head2"][0])
    wo, bo = params["out"]["W"], params["out"]["b"]

    x_pad = jnp.pad(x, ((0, 0), (0, 4)))
    wab = jnp.concatenate([wa, wb], axis=1)
    wab = jnp.pad(wab, ((0, 4), (0, 0)))
    ab = _rows_call(_prep_body, n, 2000, [x_pad], [wab], 128)

    la, lb = _edge_gather_sc(ab, src, dst, e)
    be = 3200
    bcast = [b1[None, :], ec[1][0], ec[1][1][None, :],
             ec[2][0], ec[2][1][None, :]]
    msg = pl.pallas_call(
        _edge_body,
        grid=(e // be,),
        in_specs=[pl.BlockSpec((be, 128), lambda i: (i, 0)),
                  pl.BlockSpec((be, 128), lambda i: (i, 0))]
                 + [pl.BlockSpec(a.shape, lambda i: (0, 0)) for a in bcast],
        out_specs=pl.BlockSpec((be, 64), lambda i: (i, 0)),
        out_shape=jax.ShapeDtypeStruct((e, 64), jnp.float32),
    )(la, lb, *bcast)

    x0 = jax.ops.segment_max(msg.astype(jnp.bfloat16), dst, num_segments=n)
    x0 = jnp.where(jnp.isfinite(x0), x0, jnp.bfloat16(0)).astype(jnp.float32)

    def gin(h, wbt, aff):
        tab = h if h.shape[1] == 128 else jnp.pad(h, ((0, 0), (0, 64)))
        parts = _seg_sum_sc(tab, src, dst, n, e)
        (w, bb), (g, t) = wbt, aff
        return _rows_call(_gin_body, n, 2000, [h, parts[0], parts[1]],
                          [w, bb[None, :], g[None, :], t[None, :]],
                          w.shape[1])

    x1 = gin(x0, gin_w[0], bn_aff[0])
    x2 = gin(x1, gin_w[1], bn_aff[1])
    x3 = gin(x2, gin_w[2], bn_aff[2])

    out = _rows_call(_lin_body, n, 1000, [x0, x1, x2, x3],
                     [wl[:64], wl[64:128], wl[128:256], wl[256:],
                      bl[None, :]], 1024)

    mx = jax.ops.segment_max(out, batch, num_segments=b)
    mx = jnp.where(jnp.isfinite(mx), mx, 0.0)
    cnt = jax.ops.segment_sum(jnp.ones((n,), jnp.float32), batch,
                              num_segments=b)
    mean = (jax.ops.segment_sum(out, batch, num_segments=b)
            / jnp.clip(cnt, 1.0)[:, None])
    pooled = jnp.concatenate([mx, mean], axis=1)
    pooled = jnp.pad(pooled, ((0, -b % 8), (0, 0)))

    logits = _rows_call(_head_body, pooled.shape[0], pooled.shape[0],
                        [pooled],
                        [wh1, bh1[None, :], wh2, bh2[None, :],
                         wo, bo[None, :]], 40)
    return logits[:b]


# speedup vs baseline: 1.5908x; 1.5908x over previous
import functools

import jax
import jax.numpy as jnp
from jax import lax
from jax.experimental import pallas as pl
from jax.experimental.pallas import tpu as pltpu
from jax.experimental.pallas import tpu_sc as plsc

_NC, _NS = 2, 16
_CHUNK = 80

_NPG = 100
_INV = (1.0 + 1e-5) ** -0.5


def _fold(layer):
    s = layer["gamma"] * _INV
    return layer["W"] * s[None, :], layer["b"] * s + layer["beta"]


def _dot(a, w):
    return jax.lax.dot_general(
        a, w, (((1,), (0,)), ((), ())),
        preferred_element_type=jnp.float32,
        precision=jax.lax.Precision.DEFAULT,
    )


_STRIPE = 624


def _stripe_copy(s, src_ref, dst_ref, n):
    pltpu.sync_copy(src_ref.at[pl.ds(s * _STRIPE, _STRIPE)],
                    dst_ref.at[pl.ds(s * _STRIPE, _STRIPE)])
    tail = n - _STRIPE * _NS

    @pl.when(s == 0)
    def _():
        pltpu.sync_copy(src_ref.at[pl.ds(_STRIPE * _NS, tail)],
                        dst_ref.at[pl.ds(_STRIPE * _NS, tail)])


def _seg_sum_sc(table, src, dst, n, e):
    d = table.shape[1]
    nw = _NC * _NS
    epw = e // nw
    nch = epw // _CHUNK
    mesh = plsc.VectorSubcoreMesh(core_axis_name="c", subcore_axis_name="s")

    @functools.partial(
        pl.kernel, mesh=mesh,
        out_type=jax.ShapeDtypeStruct((_NC, n, d), jnp.float32),
        scratch_types=[
            pltpu.VMEM((_CHUNK,), jnp.int32),
            pltpu.VMEM((_CHUNK,), jnp.int32),
            pltpu.VMEM((_CHUNK, d), jnp.float32),
            pltpu.VMEM_SHARED((n, d), jnp.float32),
            pltpu.SemaphoreType.DMA,
        ])
    def k(table_hbm, src_hbm, dst_hbm, zeros_hbm, out_hbm,
          src_v, dst_v, rows_v, acc_sh, sem):
        c = lax.axis_index("c")
        s = lax.axis_index("s")
        w = c * _NS + s
        _stripe_copy(s, zeros_hbm, acc_sh, n)
        plsc.subcore_barrier()

        def body(i, carry):
            base = w * epw + i * _CHUNK
            pltpu.sync_copy(src_hbm.at[pl.ds(base, _CHUNK)], src_v)
            pltpu.sync_copy(dst_hbm.at[pl.ds(base, _CHUNK)], dst_v)
            pltpu.async_copy(table_hbm.at[src_v], rows_v, sem).wait()
            pltpu.sync_copy(rows_v, acc_sh.at[dst_v], add=True)
            return carry

        lax.fori_loop(0, nch, body, 0)
        plsc.subcore_barrier()
        _stripe_copy(s, acc_sh, out_hbm.at[c], n)

    return k(table, src, dst, jnp.zeros((n, d), jnp.float32))


def _edge_gather_sc(ab_tab, src, dst, e):
    d = ab_tab.shape[1]
    nw = _NC * _NS
    epw = e // nw
    nch = epw // _CHUNK
    mesh = plsc.VectorSubcoreMesh(core_axis_name="c", subcore_axis_name="s")

    @functools.partial(
        pl.kernel, mesh=mesh,
        out_type=(jax.ShapeDtypeStruct((e, d), jnp.float32),
                  jax.ShapeDtypeStruct((e, d), jnp.float32)),
        scratch_types=[
            pltpu.VMEM((_CHUNK,), jnp.int32),
            pltpu.VMEM((_CHUNK,), jnp.int32),
            pltpu.VMEM((_CHUNK, d), jnp.float32),
            pltpu.VMEM((_CHUNK, d), jnp.float32),
            pltpu.SemaphoreType.DMA,
        ])
    def k(ab_hbm, src_hbm, dst_hbm, oa_hbm, ob_hbm,
          di_v, si_v, ra_v, rb_v, sem):
        c = lax.axis_index("c")
        s = lax.axis_index("s")
        w = c * _NS + s

        def body(i, carry):
            base = w * epw + i * _CHUNK
            pltpu.sync_copy(dst_hbm.at[pl.ds(base, _CHUNK)], di_v)
            pltpu.sync_copy(src_hbm.at[pl.ds(base, _CHUNK)], si_v)
            cpa = pltpu.async_copy(ab_hbm.at[di_v], ra_v, sem)
            cpb = pltpu.async_copy(ab_hbm.at[si_v], rb_v, sem)
            cpa.wait()
            cpb.wait()
            pltpu.sync_copy(ra_v, oa_hbm.at[pl.ds(base, _CHUNK)])
            pltpu.sync_copy(rb_v, ob_hbm.at[pl.ds(base, _CHUNK)])
            return carry

        lax.fori_loop(0, nch, body, 0)

    return k(ab_tab, src, dst)


def _prep_body(x_ref, wab_ref, out_ref):
    out_ref[...] = _dot(x_ref[...], wab_ref[...])


def _edge_body(la_ref, lb_ref, b1_ref, w2_ref, b2_ref, w3_ref, b3_ref,
               out_ref):
    h = jnp.maximum(la_ref[:, :64] + lb_ref[:, 64:] + b1_ref[...], 0.0)
    h = jnp.maximum(_dot(h, w2_ref[...]) + b2_ref[...], 0.0)
    out_ref[...] = jnp.maximum(_dot(h, w3_ref[...]) + b3_ref[...], 0.0)


def _gin_body(h_ref, a0_ref, a1_ref, w_ref, b_ref, g_ref, t_ref, out_ref):
    din = h_ref.shape[1]
    u = h_ref[...] + a0_ref[:, :din] + a1_ref[:, :din]
    y = jnp.maximum(_dot(u, w_ref[...]) + b_ref[...], 0.0)
    out_ref[...] = g_ref[...] * y + t_ref[...]


def _lin_body(x0_ref, x1_ref, x2_ref, x3_ref, w0_ref, w1_ref, w2_ref,
              w3_ref, b_ref, out_ref):
    acc = _dot(x0_ref[...], w0_ref[...])
    acc += _dot(x1_ref[...], w1_ref[...])
    acc += _dot(x2_ref[...], w2_ref[...])
    acc += _dot(x3_ref[...], w3_ref[...])
    out_ref[...] = jnp.maximum(acc + b_ref[...], 0.0)


def _head_body(p_ref, w1_ref, b1_ref, w2_ref, b2_ref, wo_ref, bo_ref,
               out_ref):
    h = jnp.maximum(_dot(p_ref[...], w1_ref[...]) + b1_ref[...], 0.0)
    h = jnp.maximum(_dot(h, w2_ref[...]) + b2_ref[...], 0.0)
    z = _dot(h, wo_ref[...]) + bo_ref[...]
    zmax = jnp.max(z, axis=-1, keepdims=True)
    ez = jnp.exp(z - zmax)
    out_ref[...] = (z - zmax) - jnp.log(jnp.sum(ez, axis=-1, keepdims=True))


def _rows_call(body, n_rows, block_rows, feat_ins, bcast_ins, out_dim):
    grid = (n_rows // block_rows,)
    in_specs = (
        [pl.BlockSpec((block_rows, a.shape[-1]), lambda i: (i, 0))
         for a in feat_ins]
        + [pl.BlockSpec(a.shape, lambda i: tuple(0 for _ in a.shape))
           for a in bcast_ins]
    )
    return pl.pallas_call(
        body,
        grid=grid,
        in_specs=in_specs,
        out_specs=pl.BlockSpec((block_rows, out_dim), lambda i: (i, 0)),
        out_shape=jax.ShapeDtypeStruct((n_rows, out_dim), jnp.float32),
    )(*feat_ins, *bcast_ins)


def kernel(x, params, edge_index, batch):
    n = x.shape[0]
    e = edge_index.shape[1]
    b = n // _NPG
    src = edge_index[0]
    dst = edge_index[1]

    ec = [_fold(l) for l in params["edgeconv"]]
    w1, b1 = ec[0]
    wa = w1[:4] - w1[4:]
    wb = w1[4:]
    gin_w = [_fold(params[k][0]) for k in ("gin1", "gin2", "gin3")]
    bn_aff = [(params[k]["gamma"] * _INV, params[k]["beta"])
              for k in ("bn1", "bn2", "bn3")]
    wl, bl = _fold(params["lin"][0])
    wh1, bh1 = _fold(params["head1"][0])
    wh2, bh2 = _fold(params["head2"][0])
    wo, bo = params["out"]["W"], params["out"]["b"]

    x_pad = jnp.pad(x, ((0, 0), (0, 4)))
    wab = jnp.concatenate([wa, wb], axis=1)
    wab = jnp.pad(wab, ((0, 4), (0, 0)))
    ab = _rows_call(_prep_body, n, 2000, [x_pad], [wab], 128)

    la, lb = _edge_gather_sc(ab, src, dst, e)
    be = 3200
    bcast = [b1[None, :], ec[1][0], ec[1][1][None, :],
             ec[2][0], ec[2][1][None, :]]
    msg = pl.pallas_call(
        _edge_body,
        grid=(e // be,),
        in_specs=[pl.BlockSpec((be, 128), lambda i: (i, 0)),
                  pl.BlockSpec((be, 128), lambda i: (i, 0))]
                 + [pl.BlockSpec(a.shape, lambda i: (0, 0)) for a in bcast],
        out_specs=pl.BlockSpec((be, 64), lambda i: (i, 0)),
        out_shape=jax.ShapeDtypeStruct((e, 64), jnp.float32),
    )(la, lb, *bcast)

    x0 = jax.ops.segment_max(msg, dst, num_segments=n)
    x0 = jnp.where(jnp.isfinite(x0), x0, 0.0)

    def gin(h, wbt, aff):
        tab = h if h.shape[1] == 128 else jnp.pad(h, ((0, 0), (0, 64)))
        parts = _seg_sum_sc(tab, src, dst, n, e)
        (w, bb), (g, t) = wbt, aff
        return _rows_call(_gin_body, n, 2000, [h, parts[0], parts[1]],
                          [w, bb[None, :], g[None, :], t[None, :]],
                          w.shape[1])

    x1 = gin(x0, gin_w[0], bn_aff[0])
    x2 = gin(x1, gin_w[1], bn_aff[1])
    x3 = gin(x2, gin_w[2], bn_aff[2])

    out = _rows_call(_lin_body, n, 1000, [x0, x1, x2, x3],
                     [wl[:64], wl[64:128], wl[128:256], wl[256:],
                      bl[None, :]], 1024)

    mx = jax.ops.segment_max(out, batch, num_segments=b)
    mx = jnp.where(jnp.isfinite(mx), mx, 0.0)
    cnt = jax.ops.segment_sum(jnp.ones((n,), jnp.float32), batch,
                              num_segments=b)
    mean = (jax.ops.segment_sum(out, batch, num_segments=b)
            / jnp.clip(cnt, 1.0)[:, None])
    pooled = jnp.concatenate([mx, mean], axis=1)
    pooled = jnp.pad(pooled, ((0, -b % 8), (0, 0)))

    logits = _rows_call(_head_body, pooled.shape[0], pooled.shape[0],
                        [pooled],
                        [wh1, bh1[None, :], wh2, bh2[None, :],
                         wo, bo[None, :]], 40)
    return logits[:b]


# speedup vs baseline: 1.7776x; 1.1175x over previous
import functools

import jax
import jax.numpy as jnp
from jax import lax
from jax.experimental import pallas as pl
from jax.experimental.pallas import tpu as pltpu
from jax.experimental.pallas import tpu_sc as plsc

_NC, _NS = 2, 16
_CHUNK = 80

_NPG = 100
_INV = (1.0 + 1e-5) ** -0.5


def _fold(layer):
    s = layer["gamma"] * _INV
    return layer["W"] * s[None, :], layer["b"] * s + layer["beta"]


def _dot(a, w):
    return jax.lax.dot_general(
        a, w, (((1,), (0,)), ((), ())),
        preferred_element_type=jnp.float32,
        precision=jax.lax.Precision.DEFAULT,
    )


_STRIPE = 624


def _stripe_copy(s, src_ref, dst_ref, n):
    pltpu.sync_copy(src_ref.at[pl.ds(s * _STRIPE, _STRIPE)],
                    dst_ref.at[pl.ds(s * _STRIPE, _STRIPE)])
    tail = n - _STRIPE * _NS

    @pl.when(s == 0)
    def _():
        pltpu.sync_copy(src_ref.at[pl.ds(_STRIPE * _NS, tail)],
                        dst_ref.at[pl.ds(_STRIPE * _NS, tail)])


def _seg_sum_sc(table, src, dst, n, e):
    d = table.shape[1]
    nw = _NC * _NS
    epw = e // nw
    nch = epw // _CHUNK
    mesh = plsc.VectorSubcoreMesh(core_axis_name="c", subcore_axis_name="s")

    @functools.partial(
        pl.kernel, mesh=mesh,
        out_type=jax.ShapeDtypeStruct((_NC, n, d), jnp.float32),
        scratch_types=[
            pltpu.VMEM((_CHUNK,), jnp.int32),
            pltpu.VMEM((_CHUNK,), jnp.int32),
            pltpu.VMEM((_CHUNK,), jnp.int32),
            pltpu.VMEM((_CHUNK,), jnp.int32),
            pltpu.VMEM((_CHUNK, d), jnp.float32),
            pltpu.VMEM((_CHUNK, d), jnp.float32),
            pltpu.SemaphoreType.DMA,
            pltpu.SemaphoreType.DMA,
            pltpu.VMEM_SHARED((n, d), jnp.float32),
        ])
    def k(table_hbm, src_hbm, dst_hbm, zeros_hbm, out_hbm,
          s0_v, d0_v, s1_v, d1_v, r0_v, r1_v, sem0, sem1, acc_sh):
        c = lax.axis_index("c")
        s = lax.axis_index("s")
        w = c * _NS + s
        _stripe_copy(s, zeros_hbm, acc_sh, n)
        plsc.subcore_barrier()

        def body(i, carry):
            base0 = w * epw + 2 * i * _CHUNK
            base1 = base0 + _CHUNK
            pltpu.sync_copy(src_hbm.at[pl.ds(base0, _CHUNK)], s0_v)
            pltpu.sync_copy(dst_hbm.at[pl.ds(base0, _CHUNK)], d0_v)
            cp0 = pltpu.async_copy(table_hbm.at[s0_v], r0_v, sem0)
            pltpu.sync_copy(src_hbm.at[pl.ds(base1, _CHUNK)], s1_v)
            pltpu.sync_copy(dst_hbm.at[pl.ds(base1, _CHUNK)], d1_v)
            cp1 = pltpu.async_copy(table_hbm.at[s1_v], r1_v, sem1)
            cp0.wait()
            pltpu.sync_copy(r0_v, acc_sh.at[d0_v], add=True)
            cp1.wait()
            pltpu.sync_copy(r1_v, acc_sh.at[d1_v], add=True)
            return carry

        lax.fori_loop(0, nch // 2, body, 0)
        if nch % 2:
            base = w * epw + (nch - 1) * _CHUNK
            pltpu.sync_copy(src_hbm.at[pl.ds(base, _CHUNK)], s0_v)
            pltpu.sync_copy(dst_hbm.at[pl.ds(base, _CHUNK)], d0_v)
            pltpu.async_copy(table_hbm.at[s0_v], r0_v, sem0).wait()
            pltpu.sync_copy(r0_v, acc_sh.at[d0_v], add=True)
        plsc.subcore_barrier()
        _stripe_copy(s, acc_sh, out_hbm.at[c], n)

    return k(table, src, dst, jnp.zeros((n, d), jnp.float32))


def _edge_gather_sc(ab_tab, src, dst, e):
    d = ab_tab.shape[1]
    nw = _NC * _NS
    epw = e // nw
    nch = epw // _CHUNK
    mesh = plsc.VectorSubcoreMesh(core_axis_name="c", subcore_axis_name="s")

    @functools.partial(
        pl.kernel, mesh=mesh,
        out_type=(jax.ShapeDtypeStruct((e, d), jnp.float32),
                  jax.ShapeDtypeStruct((e, d), jnp.float32)),
        scratch_types=[
            pltpu.VMEM((_CHUNK,), jnp.int32),
            pltpu.VMEM((_CHUNK,), jnp.int32),
            pltpu.VMEM((_CHUNK, d), jnp.float32),
            pltpu.VMEM((_CHUNK, d), jnp.float32),
            pltpu.SemaphoreType.DMA,
        ])
    def k(ab_hbm, src_hbm, dst_hbm, oa_hbm, ob_hbm,
          di_v, si_v, ra_v, rb_v, sem):
        c = lax.axis_index("c")
        s = lax.axis_index("s")
        w = c * _NS + s

        def body(i, carry):
            base = w * epw + i * _CHUNK
            pltpu.sync_copy(dst_hbm.at[pl.ds(base, _CHUNK)], di_v)
            pltpu.sync_copy(src_hbm.at[pl.ds(base, _CHUNK)], si_v)
            cpa = pltpu.async_copy(ab_hbm.at[di_v], ra_v, sem)
            cpb = pltpu.async_copy(ab_hbm.at[si_v], rb_v, sem)
            cpa.wait()
            cpb.wait()
            pltpu.sync_copy(ra_v, oa_hbm.at[pl.ds(base, _CHUNK)])
            pltpu.sync_copy(rb_v, ob_hbm.at[pl.ds(base, _CHUNK)])
            return carry

        lax.fori_loop(0, nch, body, 0)

    return k(ab_tab, src, dst)


def _prep_body(x_ref, wab_ref, out_ref):
    out_ref[...] = _dot(x_ref[...], wab_ref[...])


def _edge_body(la_ref, lb_ref, b1_ref, w2_ref, b2_ref, w3_ref, b3_ref,
               out_ref):
    h = jnp.maximum(la_ref[:, :64] + lb_ref[:, 64:] + b1_ref[...], 0.0)
    h = jnp.maximum(_dot(h, w2_ref[...]) + b2_ref[...], 0.0)
    out_ref[...] = jnp.maximum(_dot(h, w3_ref[...]) + b3_ref[...], 0.0)


def _gin_body(h_ref, a0_ref, a1_ref, w_ref, b_ref, g_ref, t_ref, out_ref):
    din = h_ref.shape[1]
    u = h_ref[...] + a0_ref[:, :din] + a1_ref[:, :din]
    y = jnp.maximum(_dot(u, w_ref[...]) + b_ref[...], 0.0)
    out_ref[...] = g_ref[...] * y + t_ref[...]


def _lin_body(x0_ref, x1_ref, x2_ref, x3_ref, w0_ref, w1_ref, w2_ref,
              w3_ref, b_ref, out_ref):
    acc = _dot(x0_ref[...], w0_ref[...])
    acc += _dot(x1_ref[...], w1_ref[...])
    acc += _dot(x2_ref[...], w2_ref[...])
    acc += _dot(x3_ref[...], w3_ref[...])
    out_ref[...] = jnp.maximum(acc + b_ref[...], 0.0)


def _head_body(p_ref, w1_ref, b1_ref, w2_ref, b2_ref, wo_ref, bo_ref,
               out_ref):
    h = jnp.maximum(_dot(p_ref[...], w1_ref[...]) + b1_ref[...], 0.0)
    h = jnp.maximum(_dot(h, w2_ref[...]) + b2_ref[...], 0.0)
    z = _dot(h, wo_ref[...]) + bo_ref[...]
    zmax = jnp.max(z, axis=-1, keepdims=True)
    ez = jnp.exp(z - zmax)
    out_ref[...] = (z - zmax) - jnp.log(jnp.sum(ez, axis=-1, keepdims=True))


def _rows_call(body, n_rows, block_rows, feat_ins, bcast_ins, out_dim):
    grid = (n_rows // block_rows,)
    in_specs = (
        [pl.BlockSpec((block_rows, a.shape[-1]), lambda i: (i, 0))
         for a in feat_ins]
        + [pl.BlockSpec(a.shape, lambda i: tuple(0 for _ in a.shape))
           for a in bcast_ins]
    )
    return pl.pallas_call(
        body,
        grid=grid,
        in_specs=in_specs,
        out_specs=pl.BlockSpec((block_rows, out_dim), lambda i: (i, 0)),
        out_shape=jax.ShapeDtypeStruct((n_rows, out_dim), jnp.float32),
    )(*feat_ins, *bcast_ins)


def kernel(x, params, edge_index, batch):
    n = x.shape[0]
    e = edge_index.shape[1]
    b = n // _NPG
    src = edge_index[0]
    dst = edge_index[1]

    ec = [_fold(l) for l in params["edgeconv"]]
    w1, b1 = ec[0]
    wa = w1[:4] - w1[4:]
    wb = w1[4:]
    gin_w = [_fold(params[k][0]) for k in ("gin1", "gin2", "gin3")]
    bn_aff = [(params[k]["gamma"] * _INV, params[k]["beta"])
              for k in ("bn1", "bn2", "bn3")]
    wl, bl = _fold(params["lin"][0])
    wh1, bh1 = _fold(params["head1"][0])
    wh2, bh2 = _fold(params["head2"][0])
    wo, bo = params["out"]["W"], params["out"]["b"]

    x_pad = jnp.pad(x, ((0, 0), (0, 4)))
    wab = jnp.concatenate([wa, wb], axis=1)
    wab = jnp.pad(wab, ((0, 4), (0, 0)))
    ab = _rows_call(_prep_body, n, 2000, [x_pad], [wab], 128)

    la, lb = _edge_gather_sc(ab, src, dst, e)
    be = 3200
    bcast = [b1[None, :], ec[1][0], ec[1][1][None, :],
             ec[2][0], ec[2][1][None, :]]
    msg = pl.pallas_call(
        _edge_body,
        grid=(e // be,),
        in_specs=[pl.BlockSpec((be, 128), lambda i: (i, 0)),
                  pl.BlockSpec((be, 128), lambda i: (i, 0))]
                 + [pl.BlockSpec(a.shape, lambda i: (0, 0)) for a in bcast],
        out_specs=pl.BlockSpec((be, 64), lambda i: (i, 0)),
        out_shape=jax.ShapeDtypeStruct((e, 64), jnp.float32),
    )(la, lb, *bcast)

    x0 = jax.ops.segment_max(msg, dst, num_segments=n)
    x0 = jnp.where(jnp.isfinite(x0), x0, 0.0)

    def gin(h, wbt, aff):
        tab = h if h.shape[1] == 128 else jnp.pad(h, ((0, 0), (0, 64)))
        parts = _seg_sum_sc(tab, src, dst, n, e)
        (w, bb), (g, t) = wbt, aff
        return _rows_call(_gin_body, n, 2000, [h, parts[0], parts[1]],
                          [w, bb[None, :], g[None, :], t[None, :]],
                          w.shape[1])

    x1 = gin(x0, gin_w[0], bn_aff[0])
    x2 = gin(x1, gin_w[1], bn_aff[1])
    x3 = gin(x2, gin_w[2], bn_aff[2])

    out = _rows_call(_lin_body, n, 1000, [x0, x1, x2, x3],
                     [wl[:64], wl[64:128], wl[128:256], wl[256:],
                      bl[None, :]], 1024)

    mx = jax.ops.segment_max(out, batch, num_segments=b)
    mx = jnp.where(jnp.isfinite(mx), mx, 0.0)
    cnt = jax.ops.segment_sum(jnp.ones((n,), jnp.float32), batch,
                              num_segments=b)
    mean = (jax.ops.segment_sum(out, batch, num_segments=b)
            / jnp.clip(cnt, 1.0)[:, None])
    pooled = jnp.concatenate([mx, mean], axis=1)
    pooled = jnp.pad(pooled, ((0, -b % 8), (0, 0)))

    logits = _rows_call(_head_body, pooled.shape[0], pooled.shape[0],
                        [pooled],
                        [wh1, bh1[None, :], wh2, bh2[None, :],
                         wo, bo[None, :]], 40)
    return logits[:b]
